# Initial kernel scaffold; baseline (speedup 1.0000x reference)
import jax
import jax.numpy as jnp
from jax.experimental import pallas as pl


def kernel(x_i, x_j, edge_attribute, senders, receivers, Ws, Wt, We, attn):
    raise NotImplementedError("write your pallas kernel here")


# speedup vs baseline: 11.2958x; 11.2958x over previous
import functools

import jax
import jax.numpy as jnp
from jax.experimental import pallas as pl
from jax.experimental.pallas import tpu as pltpu

N_NODES = 10000
N_EDGES = 320000
IN_CH = 128
HEADS = 4
OUT_CH = 32

_B = 512


def _phase1_body(xi, xj, ea, ws, wt, we, amat, t_out, ez_out):
    t = jnp.dot(xj[...], wt[...], preferred_element_type=jnp.float32)
    u = (
        jnp.dot(xi[...], ws[...], preferred_element_type=jnp.float32)
        + jnp.dot(ea[...], we[...], preferred_element_type=jnp.float32)
        + t
    )
    u = jnp.where(u >= 0.0, u, 0.01 * u)
    z = jnp.dot(u, amat[...], preferred_element_type=jnp.float32)
    t_out[...] = t
    ez_out[...] = jnp.exp(z)


def _phase4_body(t, a, m_out):
    acc = jnp.zeros((t.shape[0], OUT_CH), jnp.float32)
    for h in range(HEADS):
        acc = acc + t[:, h * OUT_CH:(h + 1) * OUT_CH] * a[:, h:h + 1]
    m_out[...] = 0.25 * acc


def kernel(x_i, x_j, edge_attribute, senders, receivers, Ws, Wt, We, attn):
    attn_flat = attn.reshape(HEADS * OUT_CH)
    head_id = jnp.arange(HEADS * OUT_CH, dtype=jnp.int32) // OUT_CH
    amat = jnp.where(
        head_id[:, None] == jnp.arange(HEADS, dtype=jnp.int32)[None, :],
        attn_flat[:, None],
        0.0,
    ).astype(jnp.float32)

    grid = (N_EDGES // _B,)
    row_spec = pl.BlockSpec((_B, IN_CH), lambda i: (i, 0))
    w_spec = pl.BlockSpec((IN_CH, IN_CH), lambda i: (0, 0))
    t, ez = pl.pallas_call(
        _phase1_body,
        grid=grid,
        in_specs=[row_spec, row_spec, row_spec, w_spec, w_spec, w_spec,
                  pl.BlockSpec((IN_CH, HEADS), lambda i: (0, 0))],
        out_specs=[pl.BlockSpec((_B, IN_CH), lambda i: (i, 0)),
                   pl.BlockSpec((_B, HEADS), lambda i: (i, 0))],
        out_shape=[jax.ShapeDtypeStruct((N_EDGES, IN_CH), jnp.float32),
                   jax.ShapeDtypeStruct((N_EDGES, HEADS), jnp.float32)],
    )(x_i, x_j, edge_attribute, Ws, Wt, We, amat)

    denom = jax.ops.segment_sum(ez, senders, num_segments=N_NODES)
    a = ez / denom[senders]

    m = pl.pallas_call(
        _phase4_body,
        grid=grid,
        in_specs=[pl.BlockSpec((_B, IN_CH), lambda i: (i, 0)),
                  pl.BlockSpec((_B, HEADS), lambda i: (i, 0))],
        out_specs=pl.BlockSpec((_B, OUT_CH), lambda i: (i, 0)),
        out_shape=jax.ShapeDtypeStruct((N_EDGES, OUT_CH), jnp.float32),
    )(t, a)

    aggr = jax.ops.segment_sum(m, receivers, num_segments=N_NODES)
    return (aggr, m)
